# baseline (speedup 1.0000x reference)
import math

import jax
import jax.numpy as jnp
from jax.experimental import pallas as pl

RATIO = 0.5


def kernel(feature, edge_index, W, b):
    N = feature.shape[0]
    src = edge_index[0]
    dst = edge_index[1]
    ones = jnp.ones((src.shape[0],), dtype=feature.dtype)
    out_deg = jax.ops.segment_sum(ones, src, num_segments=N)
    in_deg = jax.ops.segment_sum(ones, dst, num_segments=N)
    norm_src = jnp.power(jnp.maximum(out_deg, 1.0), -0.5)
    norm_dst = jnp.power(jnp.maximum(in_deg, 1.0), -0.5)
    h = feature * norm_src[:, None]
    h = h @ W
    msg = jnp.take(h, src, axis=0)
    agg = jax.ops.segment_sum(msg, dst, num_segments=N)
    score = (agg * norm_dst[:, None] + b).squeeze(-1)
    k = int(math.ceil(RATIO * N))
    topk_vals, perm = jax.lax.top_k(score, k)
    feat_out = jnp.take(feature, perm, axis=0) * jnp.tanh(jnp.take(score, perm))[:, None]
    score_soft = jax.nn.softmax(score)
    next_batch_num_nodes = jnp.array([k], dtype=jnp.int32)
    return feat_out, perm, score_soft, next_batch_num_nodes


# speedup vs baseline: 1.7904x; 1.7904x over previous
import functools
import math

import jax
import jax.numpy as jnp
from jax import lax
from jax.experimental import pallas as pl
from jax.experimental.pallas import tpu as pltpu
from jax.experimental.pallas import tpu_sc as plsc

N = 10000
E = 160000
D = 256
K = 5000
NW = 32
N2 = 10240
NP = 10496
ZW = NP // 16
EW = E // NW
EC = 40
KP = 5120

_mesh = plsc.VectorSubcoreMesh(core_axis_name="c", subcore_axis_name="s")
_f32 = jnp.float32
_i32 = jnp.int32


def _fill(ref, n, val, dtype):
    for j in range(n // 16):
        ref[pl.ds(j * 16, 16)] = jnp.full((16,), val, dtype)


@functools.partial(
    pl.kernel,
    out_type=jax.ShapeDtypeStruct((2 * NP,), _f32),
    mesh=_mesh,
    scratch_types=[
        pltpu.VMEM_SHARED((NP,), _f32),
        pltpu.VMEM((EC, 128), _i32),
        pltpu.VMEM((ZW,), _f32),
        pltpu.VMEM((ZW,), _f32),
        pltpu.VMEM((128,), _f32),
    ],
)
def _sc_degree(src_hbm, out_hbm, acc_sh, idx_v, zbuf, wo_v, ones_v):
    cid = lax.axis_index("c")
    sid = lax.axis_index("s")
    w = sid * 2 + cid
    _fill(zbuf, ZW, 0.0, _f32)
    _fill(ones_v, 128, 1.0, _f32)
    pltpu.sync_copy(zbuf, acc_sh.at[pl.ds(sid * ZW, ZW)])
    plsc.subcore_barrier()
    pltpu.sync_copy(src_hbm.at[w], idx_v)
    for c in range(EC):
        pltpu.sync_copy(ones_v, acc_sh.at[idx_v.at[c]], add=True)
    plsc.subcore_barrier()
    pltpu.sync_copy(acc_sh.at[pl.ds(sid * ZW, ZW)], wo_v)
    pltpu.sync_copy(wo_v, out_hbm.at[pl.ds(cid * NP + sid * ZW, ZW)])


@functools.partial(
    pl.kernel,
    out_type=(
        jax.ShapeDtypeStruct((2 * NP,), _f32),
        jax.ShapeDtypeStruct((2 * NP,), _f32),
    ),
    mesh=_mesh,
    scratch_types=[
        pltpu.VMEM_SHARED((NP,), _f32),
        pltpu.VMEM_SHARED((NP,), _f32),
        pltpu.VMEM_SHARED((NP,), _f32),
        pltpu.VMEM((EC, 128), _i32),
        pltpu.VMEM((EC, 128), _i32),
        pltpu.VMEM((EC, 128), _f32),
        pltpu.VMEM((ZW,), _f32),
        pltpu.VMEM((ZW,), _f32),
        pltpu.VMEM((128,), _f32),
        pltpu.SemaphoreType.DMA,
    ],
)
def _sc_msg(h_hbm, src_hbm, dst_hbm, agg_out, deg_out,
            h_sh, agg_sh, deg_sh, sidx_v, didx_v, msg_v, zbuf, wo_v, ones_v, sem):
    cid = lax.axis_index("c")
    sid = lax.axis_index("s")
    w = sid * 2 + cid
    _fill(zbuf, ZW, 0.0, _f32)
    _fill(ones_v, 128, 1.0, _f32)
    pltpu.sync_copy(zbuf, agg_sh.at[pl.ds(sid * ZW, ZW)])
    pltpu.sync_copy(zbuf, deg_sh.at[pl.ds(sid * ZW, ZW)])

    pltpu.sync_copy(h_hbm.at[pl.ds(sid * ZW, ZW)], wo_v)
    pltpu.sync_copy(wo_v, h_sh.at[pl.ds(sid * ZW, ZW)])
    plsc.subcore_barrier()
    pltpu.sync_copy(src_hbm.at[w], sidx_v)
    pltpu.sync_copy(dst_hbm.at[w], didx_v)
    for c in range(EC):
        pltpu.async_copy(h_sh.at[sidx_v.at[c]], msg_v.at[c], sem).wait()
        pltpu.sync_copy(msg_v.at[c], agg_sh.at[didx_v.at[c]], add=True)
        pltpu.sync_copy(ones_v, deg_sh.at[didx_v.at[c]], add=True)
    plsc.subcore_barrier()
    pltpu.sync_copy(agg_sh.at[pl.ds(sid * ZW, ZW)], wo_v)
    pltpu.sync_copy(wo_v, agg_out.at[pl.ds(cid * NP + sid * ZW, ZW)])
    pltpu.sync_copy(deg_sh.at[pl.ds(sid * ZW, ZW)], wo_v)
    pltpu.sync_copy(wo_v, deg_out.at[pl.ds(cid * NP + sid * ZW, ZW)])


@functools.partial(
    pl.kernel,
    out_type=jax.ShapeDtypeStruct((N2,), _i32),
    mesh=_mesh,
    scratch_types=[
        pltpu.VMEM((4, 80), _i32),
        pltpu.VMEM((80,), _i32),
    ],
)
def _sc_permscat(rank_hbm, perm_out, idx_v, val_v):
    cid = lax.axis_index("c")
    sid = lax.axis_index("s")
    w = sid * 2 + cid
    pltpu.sync_copy(rank_hbm.at[w], idx_v)
    for c in range(4):
        base = w * 320 + c * 80
        for j in range(5):
            val_v[pl.ds(j * 16, 16)] = lax.iota(_i32, 16) + (base + j * 16)
        pltpu.sync_copy(val_v, perm_out.at[idx_v.at[c]])


@functools.partial(
    pl.kernel,
    out_type=(
        jax.ShapeDtypeStruct((KP, D), _f32),
        jax.ShapeDtypeStruct((KP,), _f32),
    ),
    mesh=_mesh,
    scratch_types=[
        pltpu.VMEM((2, 80), _i32),
        pltpu.VMEM((80, D), _f32),
        pltpu.VMEM((80,), _f32),
        pltpu.SemaphoreType.DMA,
    ],
)
def _sc_gather(perm_hbm, feat_hbm, t_hbm, rows_out, tp_out, idx_v, rows_v, tv_v, sem):
    cid = lax.axis_index("c")
    sid = lax.axis_index("s")
    w = sid * 2 + cid
    pltpu.sync_copy(perm_hbm.at[w], idx_v)
    for c in range(2):
        base = w * 160 + c * 80
        pltpu.async_copy(feat_hbm.at[idx_v.at[c]], rows_v, sem).wait()
        pltpu.sync_copy(rows_v, rows_out.at[pl.ds(base, 80), :])
        pltpu.async_copy(t_hbm.at[idx_v.at[c]], tv_v, sem).wait()
        pltpu.sync_copy(tv_v, tp_out.at[pl.ds(base, 80)])


def _tc_matvec_body(od_ref, f_ref, w_ref, h_ref):
    deg = od_ref[0] + od_ref[1]
    ns = lax.rsqrt(jnp.maximum(deg, 1.0))
    ns_flat = ns.reshape(NP)[:N]
    x = f_ref[...] * ns_flat[:, None]
    h_ref[pl.ds(0, N)] = jnp.dot(x, w_ref[...])[:, 0]


def _tc_matvec(od_part, feature, w_pad):
    return pl.pallas_call(
        _tc_matvec_body,
        out_shape=jax.ShapeDtypeStruct((NP,), _f32),
    )(od_part, feature, w_pad)


def _tc_score_body(ap_ref, ip_ref, b2_ref, rank_ref, soft_ref, t_ref, m_ref):
    agg = ap_ref[0, :80, :] + ap_ref[1, :80, :]
    indeg = ip_ref[0, :80, :] + ip_ref[1, :80, :]
    nd = lax.rsqrt(jnp.maximum(indeg, 1.0))
    score = agg * nd + b2_ref[...]
    row = lax.broadcasted_iota(_i32, (80, 128), 0)
    col = lax.broadcasted_iota(_i32, (80, 128), 1)
    real = (row * 128 + col) < N
    u = lax.bitcast_convert_type(score, _i32)
    m = u ^ (lax.shift_right_arithmetic(u, 31) & jnp.int32(0x7FFFFFFF))
    m = jnp.where(real, m, jnp.int32(-(2 ** 31)))
    m_ref[...] = m

    tri = (lax.broadcasted_iota(_i32, (128, 128), 1)
           < lax.broadcasted_iota(_i32, (128, 128), 0))

    def outer(bi, _):
        mi = m_ref[pl.ds(bi, 1), :]
        mi_c = jnp.transpose(mi)

        def inner_gt(r, acc):
            mj = m_ref[pl.ds(r, 1), :]
            return acc + (mj > mi_c).astype(_i32)

        acc = lax.fori_loop(0, 80, inner_gt, jnp.zeros((128, 128), _i32))

        def inner_eq(r, acc):
            mj = m_ref[pl.ds(r, 1), :]
            return acc + (mj == mi_c).astype(_i32)

        acc = lax.fori_loop(0, bi, inner_eq, acc)
        mj_d = mi
        acc = acc + ((mj_d == mi_c) & tri).astype(_i32)
        rank_ref[pl.ds(bi * 128, 128)] = jnp.sum(acc, axis=1)
        return 0

    lax.fori_loop(0, 80, outer, 0)

    s_soft = jnp.where(real, score, -jnp.inf)
    mx = jnp.max(s_soft)
    e = jnp.exp(s_soft - mx)
    soft_ref[...] = (e / jnp.sum(e)).reshape(N2)
    t_ref[...] = jnp.tanh(score).reshape(N2)


def _tc_score(agg_part, ideg_part, b2):
    return pl.pallas_call(
        _tc_score_body,
        out_shape=(
            jax.ShapeDtypeStruct((N2,), _i32),
            jax.ShapeDtypeStruct((N2,), _f32),
            jax.ShapeDtypeStruct((N2,), _f32),
        ),
        scratch_shapes=[pltpu.VMEM((80, 128), _i32)],
    )(agg_part, ideg_part, b2)


def _tc_scale_body(fr_ref, tp_ref, out_ref):
    out_ref[...] = fr_ref[...] * tp_ref[...][:, None]


def _tc_scale(feat_raw, t_perm):
    return pl.pallas_call(
        _tc_scale_body,
        out_shape=jax.ShapeDtypeStruct((KP, D), _f32),
    )(feat_raw, t_perm)


def kernel(feature, edge_index, W, b):
    src = edge_index[0]
    dst = edge_index[1]
    pad_idx = (N2 + (jnp.arange(EC * 128 - EW, dtype=_i32) % (NP - N2)))
    pad_blk = jnp.broadcast_to(pad_idx, (NW, EC * 128 - EW))
    src_p = jnp.concatenate([src.reshape(NW, EW), pad_blk], axis=1).reshape(NW, EC, 128)
    dst_p = jnp.concatenate([dst.reshape(NW, EW), pad_blk], axis=1).reshape(NW, EC, 128)

    w_pad = jnp.pad(W, ((0, 0), (0, 127)))
    b2 = jnp.broadcast_to(b.reshape(1, 1), (80, 128))

    od_part = _sc_degree(src_p)
    h = _tc_matvec(od_part.reshape(2, 82, 128), feature, w_pad)
    agg_part, ideg_part = _sc_msg(h, src_p, dst_p)
    rank, soft, t = _tc_score(agg_part.reshape(2, 82, 128),
                              ideg_part.reshape(2, 82, 128), b2)
    perm_full = _sc_permscat(rank.reshape(NW, 4, 80))
    feat_raw, t_perm = _sc_gather(perm_full[:KP].reshape(NW, 2, 80), feature, t)
    feat_out = _tc_scale(feat_raw, t_perm)

    return (
        feat_out[:K],
        perm_full[:K],
        soft[:N],
        jnp.array([K], dtype=jnp.int32),
    )


# speedup vs baseline: 3.4817x; 1.9446x over previous
import functools
import math

import jax
import jax.numpy as jnp
from jax import lax
from jax.experimental import pallas as pl
from jax.experimental.pallas import tpu as pltpu
from jax.experimental.pallas import tpu_sc as plsc

N = 10000
E = 160000
D = 256
K = 5000
NW = 32
N2 = 10240
NP = 10496
ZW = NP // 16
EW = E // NW
EC = 40
KP = 5120

_mesh = plsc.VectorSubcoreMesh(core_axis_name="c", subcore_axis_name="s")
_f32 = jnp.float32
_i32 = jnp.int32


def _fill(ref, n, val, dtype):
    for j in range(n // 16):
        ref[pl.ds(j * 16, 16)] = jnp.full((16,), val, dtype)


@functools.partial(
    pl.kernel,
    out_type=jax.ShapeDtypeStruct((2 * NP,), _f32),
    mesh=_mesh,
    scratch_types=[
        pltpu.VMEM_SHARED((NP,), _f32),
        pltpu.VMEM((EC, 128), _i32),
        pltpu.VMEM((ZW,), _f32),
        pltpu.VMEM((ZW,), _f32),
        pltpu.VMEM((128,), _f32),
    ],
)
def _sc_degree(src_hbm, out_hbm, acc_sh, idx_v, zbuf, wo_v, ones_v):
    cid = lax.axis_index("c")
    sid = lax.axis_index("s")
    w = sid * 2 + cid
    _fill(zbuf, ZW, 0.0, _f32)
    _fill(ones_v, 128, 1.0, _f32)
    pltpu.sync_copy(zbuf, acc_sh.at[pl.ds(sid * ZW, ZW)])
    plsc.subcore_barrier()
    pltpu.sync_copy(src_hbm.at[w], idx_v)
    for c in range(EC):
        pltpu.sync_copy(ones_v, acc_sh.at[idx_v.at[c]], add=True)
    plsc.subcore_barrier()
    pltpu.sync_copy(acc_sh.at[pl.ds(sid * ZW, ZW)], wo_v)
    pltpu.sync_copy(wo_v, out_hbm.at[pl.ds(cid * NP + sid * ZW, ZW)])


@functools.partial(
    pl.kernel,
    out_type=(
        jax.ShapeDtypeStruct((2 * NP,), _f32),
        jax.ShapeDtypeStruct((2 * NP,), _f32),
    ),
    mesh=_mesh,
    scratch_types=[
        pltpu.VMEM_SHARED((NP,), _f32),
        pltpu.VMEM_SHARED((NP,), _f32),
        pltpu.VMEM_SHARED((NP,), _f32),
        pltpu.VMEM((EC, 128), _i32),
        pltpu.VMEM((EC, 128), _i32),
        pltpu.VMEM((EC, 128), _f32),
        pltpu.VMEM((ZW,), _f32),
        pltpu.VMEM((ZW,), _f32),
        pltpu.VMEM((128,), _f32),
        pltpu.SemaphoreType.DMA,
    ],
)
def _sc_msg(h_hbm, src_hbm, dst_hbm, agg_out, deg_out,
            h_sh, agg_sh, deg_sh, sidx_v, didx_v, msg_v, zbuf, wo_v, ones_v, sem):
    cid = lax.axis_index("c")
    sid = lax.axis_index("s")
    w = sid * 2 + cid
    _fill(zbuf, ZW, 0.0, _f32)
    _fill(ones_v, 128, 1.0, _f32)
    pltpu.sync_copy(zbuf, agg_sh.at[pl.ds(sid * ZW, ZW)])
    pltpu.sync_copy(zbuf, deg_sh.at[pl.ds(sid * ZW, ZW)])

    pltpu.sync_copy(h_hbm.at[pl.ds(sid * ZW, ZW)], wo_v)
    pltpu.sync_copy(wo_v, h_sh.at[pl.ds(sid * ZW, ZW)])
    plsc.subcore_barrier()
    pltpu.sync_copy(src_hbm.at[w], sidx_v)
    pltpu.sync_copy(dst_hbm.at[w], didx_v)
    for c in range(EC):
        pltpu.async_copy(h_sh.at[sidx_v.at[c]], msg_v.at[c], sem).wait()
        pltpu.sync_copy(msg_v.at[c], agg_sh.at[didx_v.at[c]], add=True)
        pltpu.sync_copy(ones_v, deg_sh.at[didx_v.at[c]], add=True)
    plsc.subcore_barrier()
    pltpu.sync_copy(agg_sh.at[pl.ds(sid * ZW, ZW)], wo_v)
    pltpu.sync_copy(wo_v, agg_out.at[pl.ds(cid * NP + sid * ZW, ZW)])
    pltpu.sync_copy(deg_sh.at[pl.ds(sid * ZW, ZW)], wo_v)
    pltpu.sync_copy(wo_v, deg_out.at[pl.ds(cid * NP + sid * ZW, ZW)])


@functools.partial(
    pl.kernel,
    out_type=jax.ShapeDtypeStruct((N2,), _i32),
    mesh=_mesh,
    scratch_types=[
        pltpu.VMEM((4, 80), _i32),
        pltpu.VMEM((80,), _i32),
    ],
)
def _sc_permscat(rank_hbm, perm_out, idx_v, val_v):
    cid = lax.axis_index("c")
    sid = lax.axis_index("s")
    w = sid * 2 + cid
    pltpu.sync_copy(rank_hbm.at[w], idx_v)
    for c in range(4):
        base = w * 320 + c * 80
        for j in range(5):
            val_v[pl.ds(j * 16, 16)] = lax.iota(_i32, 16) + (base + j * 16)
        pltpu.sync_copy(val_v, perm_out.at[idx_v.at[c]])


@functools.partial(
    pl.kernel,
    out_type=(
        jax.ShapeDtypeStruct((KP, D), _f32),
        jax.ShapeDtypeStruct((KP,), _f32),
    ),
    mesh=_mesh,
    scratch_types=[
        pltpu.VMEM((2, 80), _i32),
        pltpu.VMEM((80, D), _f32),
        pltpu.VMEM((80,), _f32),
        pltpu.SemaphoreType.DMA,
    ],
)
def _sc_gather(perm_hbm, feat_hbm, t_hbm, rows_out, tp_out, idx_v, rows_v, tv_v, sem):
    cid = lax.axis_index("c")
    sid = lax.axis_index("s")
    w = sid * 2 + cid
    pltpu.sync_copy(perm_hbm.at[w], idx_v)
    for c in range(2):
        base = w * 160 + c * 80
        pltpu.async_copy(feat_hbm.at[idx_v.at[c]], rows_v, sem).wait()
        pltpu.sync_copy(rows_v, rows_out.at[pl.ds(base, 80), :])
        pltpu.async_copy(t_hbm.at[idx_v.at[c]], tv_v, sem).wait()
        pltpu.sync_copy(tv_v, tp_out.at[pl.ds(base, 80)])


def _tc_matvec_body(od_ref, f_ref, w_ref, h_ref):
    deg = od_ref[0] + od_ref[1]
    ns = lax.rsqrt(jnp.maximum(deg, 1.0))
    ns_flat = ns.reshape(NP)[:N]
    x = f_ref[...] * ns_flat[:, None]
    h_ref[pl.ds(0, N)] = jnp.dot(x, w_ref[...])[:, 0]


def _tc_matvec(od_part, feature, w_pad):
    return pl.pallas_call(
        _tc_matvec_body,
        out_shape=jax.ShapeDtypeStruct((NP,), _f32),
    )(od_part, feature, w_pad)


def _tc_score_body(ap_ref, ip_ref, b2_ref, rank_ref, soft_ref, t_ref, m_ref):
    agg = ap_ref[0, :80, :] + ap_ref[1, :80, :]
    indeg = ip_ref[0, :80, :] + ip_ref[1, :80, :]
    nd = lax.rsqrt(jnp.maximum(indeg, 1.0))
    score = agg * nd + b2_ref[...]
    row = lax.broadcasted_iota(_i32, (80, 128), 0)
    col = lax.broadcasted_iota(_i32, (80, 128), 1)
    real = (row * 128 + col) < N
    u = lax.bitcast_convert_type(score, _i32)
    m = u ^ (lax.shift_right_arithmetic(u, 31) & jnp.int32(0x7FFFFFFF))
    m = jnp.where(real, m, jnp.int32(-(2 ** 31)))
    m_ref[...] = m

    j0 = (lax.broadcasted_iota(_i32, (8, 128), 0) * 128
          + lax.broadcasted_iota(_i32, (8, 128), 1))
    diff = j0[None, :, :] - lax.broadcasted_iota(_i32, (128, 8, 128), 0)

    def outer(bi, _):
        mi3 = jnp.transpose(m_ref[pl.ds(bi, 1), :]).reshape(128, 1, 1)

        def inner_gt(jc, acc):
            mj = m_ref[pl.ds(jc * 8, 8), :][None, :, :]
            return acc + jnp.sum((mj > mi3).astype(_i32), axis=1)

        acc = lax.fori_loop(0, 10, inner_gt, jnp.zeros((128, 128), _i32))

        def inner_eq(jc, acc):
            mj = m_ref[pl.ds(jc * 8, 8), :][None, :, :]
            lt = diff < (bi * 128 - jc * 1024)
            return acc + jnp.sum(((mj == mi3) & lt).astype(_i32), axis=1)

        acc = lax.fori_loop(0, bi // 8 + 1, inner_eq, acc)
        rank_ref[pl.ds(bi * 128, 128)] = jnp.sum(acc, axis=1)
        return 0

    lax.fori_loop(0, 80, outer, 0)

    s_soft = jnp.where(real, score, -jnp.inf)
    mx = jnp.max(s_soft)
    e = jnp.exp(s_soft - mx)
    soft_ref[...] = (e / jnp.sum(e)).reshape(N2)
    t_ref[...] = jnp.tanh(score).reshape(N2)


def _tc_score(agg_part, ideg_part, b2):
    return pl.pallas_call(
        _tc_score_body,
        out_shape=(
            jax.ShapeDtypeStruct((N2,), _i32),
            jax.ShapeDtypeStruct((N2,), _f32),
            jax.ShapeDtypeStruct((N2,), _f32),
        ),
        scratch_shapes=[pltpu.VMEM((80, 128), _i32)],
    )(agg_part, ideg_part, b2)


def _tc_scale_body(fr_ref, tp_ref, out_ref):
    out_ref[...] = fr_ref[...] * tp_ref[...][:, None]


def _tc_scale(feat_raw, t_perm):
    return pl.pallas_call(
        _tc_scale_body,
        out_shape=jax.ShapeDtypeStruct((KP, D), _f32),
    )(feat_raw, t_perm)


def kernel(feature, edge_index, W, b):
    src = edge_index[0]
    dst = edge_index[1]
    pad_idx = (N2 + (jnp.arange(EC * 128 - EW, dtype=_i32) % (NP - N2)))
    pad_blk = jnp.broadcast_to(pad_idx, (NW, EC * 128 - EW))
    src_p = jnp.concatenate([src.reshape(NW, EW), pad_blk], axis=1).reshape(NW, EC, 128)
    dst_p = jnp.concatenate([dst.reshape(NW, EW), pad_blk], axis=1).reshape(NW, EC, 128)

    w_pad = jnp.pad(W, ((0, 0), (0, 127)))
    b2 = jnp.broadcast_to(b.reshape(1, 1), (80, 128))

    od_part = _sc_degree(src_p)
    h = _tc_matvec(od_part.reshape(2, 82, 128), feature, w_pad)
    agg_part, ideg_part = _sc_msg(h, src_p, dst_p)
    rank, soft, t = _tc_score(agg_part.reshape(2, 82, 128),
                              ideg_part.reshape(2, 82, 128), b2)
    perm_full = _sc_permscat(rank.reshape(NW, 4, 80))
    feat_raw, t_perm = _sc_gather(perm_full[:KP].reshape(NW, 2, 80), feature, t)
    feat_out = _tc_scale(feat_raw, t_perm)

    return (
        feat_out[:K],
        perm_full[:K],
        soft[:N],
        jnp.array([K], dtype=jnp.int32),
    )
